# baseline (speedup 1.0000x reference)
import functools
import math

import jax
import jax.numpy as jnp
from jax import lax
from jax.experimental import pallas as pl
from jax.experimental.pallas import tpu as pltpu
from jax.experimental.pallas import tpu_sc as plsc

NODE_SIZE = 128
NUM_RADIAL = 5
CUTOFF = 5.0

_NC = 2
_NS = 16

EB = 1000
NB = 1000


def _silu(x):
    return x * jax.nn.sigmoid(x)


def _softmax(x):
    m = jnp.max(x, axis=1, keepdims=True)
    e = jnp.exp(x - m)
    return e / jnp.sum(e, axis=1, keepdims=True)


def _edge_body(gsrc, gdst, gvec, dist, ed0, ed1, ed2,
               W1s, W1d, b1, W2, b2, sfW, sfb,
               mss, envc, vv0, vv1, vv2, ev0, ev1, ev2):
    h = _silu(jnp.dot(gsrc[...], W1s[...], preferred_element_type=jnp.float32)
              + jnp.dot(gdst[...], W1d[...], preferred_element_type=jnp.float32)
              + b1[...])
    mg = jnp.dot(h, W2[...], preferred_element_type=jnp.float32) + b2[...]
    d = dist[...]
    s = jnp.broadcast_to(sfb[...], mg.shape)
    inv_d = 1.0 / d
    for k in range(NUM_RADIAL):
        fk = (k + 1) * math.pi / CUTOFF
        s = s + (jnp.sin(d * fk) * inv_d) * sfW[k:k + 1, :]
    cut = jnp.where(d < CUTOFF, 0.5 * (jnp.cos(d * (math.pi / CUTOFF)) + 1.0), 0.0)
    mg = mg * (s * cut)
    gate_vv = _softmax(mg[:, 0:128])
    gate_ev = _softmax(mg[:, 128:256])
    mss[...] = mg[:, 256:384]
    envc[...] = mg[:, 384:512]
    gv = gvec[...]
    vv0[...] = gate_vv * gv[:, 0:128]
    vv1[...] = gate_vv * gv[:, 128:256]
    vv2[...] = gate_vv * gv[:, 256:384]
    ev0[...] = gate_ev * ed0[...]
    ev1[...] = gate_ev * ed1[...]
    ev2[...] = gate_ev * ed2[...]


def _edge_compute(gsrc, gdst, gvec, dist2d, ed0, ed1, ed2, params):
    E = gsrc.shape[0]
    grid = E // EB
    W1s = params['sg_W1'][0:128]
    W1d = params['sg_W1'][128:256]
    b1 = params['sg_b1'][None, :]
    b2 = params['sg_b2'][None, :]
    sfb = params['sf_b'][None, :]
    eb = lambda d: pl.BlockSpec((EB, d), lambda i: (i, 0))
    wb = lambda a: pl.BlockSpec(a.shape, lambda i: tuple(0 for _ in a.shape))
    out_shapes = tuple(jax.ShapeDtypeStruct((E, 128), jnp.float32)
                       for _ in range(8))
    return pl.pallas_call(
        _edge_body,
        grid=(grid,),
        in_specs=[eb(128), eb(128), eb(384), eb(1), eb(1), eb(1), eb(1),
                  wb(W1s), wb(W1d), wb(b1), wb(params['sg_W2']), wb(b2),
                  wb(params['sf_W']), wb(sfb)],
        out_specs=tuple(eb(128) for _ in range(8)),
        out_shape=out_shapes,
    )(gsrc, gdst, gvec, dist2d, ed0, ed1, ed2,
      W1s, W1d, b1, params['sg_W2'], b2, params['sf_W'], sfb)


def _gather(ns, nv2d, src, dst):
    N = ns.shape[0]
    E = src.shape[0]
    NW = _NC * _NS
    ept = E // NW
    nfull = ept // 128
    tail = ept - nfull * 128
    assert ept * NW == E and tail % 8 == 0

    mesh = plsc.VectorSubcoreMesh(core_axis_name="c", subcore_axis_name="s")

    @functools.partial(
        pl.kernel,
        out_type=(jax.ShapeDtypeStruct((E, 128), jnp.float32),
                  jax.ShapeDtypeStruct((E, 128), jnp.float32),
                  jax.ShapeDtypeStruct((E, 384), jnp.float32)),
        mesh=mesh,
        scratch_types=[
            pltpu.VMEM((128,), jnp.int32),
            pltpu.VMEM((128,), jnp.int32),
            pltpu.VMEM((128, 128), jnp.float32),
            pltpu.VMEM((128, 128), jnp.float32),
            pltpu.VMEM((128, 384), jnp.float32),
            pltpu.VMEM((max(tail, 8),), jnp.int32),
            pltpu.VMEM((max(tail, 8),), jnp.int32),
            pltpu.VMEM((max(tail, 8), 128), jnp.float32),
            pltpu.VMEM((max(tail, 8), 128), jnp.float32),
            pltpu.VMEM((max(tail, 8), 384), jnp.float32),
            pltpu.SemaphoreType.DMA,
        ],
    )
    def k(ns_h, nv_h, src_h, dst_h, gs_h, gd_h, gv_h,
          sidx, didx, bs, bd, bv, tsidx, tdidx, tbs, tbd, tbv, sem):
        cid = lax.axis_index("c")
        sid = lax.axis_index("s")
        ebase = (cid * _NS + sid) * ept

        def chunk(base, cn, sidx, didx, bs, bd, bv):
            pltpu.sync_copy(src_h.at[pl.ds(base, cn)], sidx)
            pltpu.sync_copy(dst_h.at[pl.ds(base, cn)], didx)
            a = pltpu.async_copy(ns_h.at[sidx], bs, sem)
            b = pltpu.async_copy(ns_h.at[didx], bd, sem)
            c = pltpu.async_copy(nv_h.at[didx], bv, sem)
            a.wait()
            b.wait()
            c.wait()
            pltpu.sync_copy(bs, gs_h.at[pl.ds(base, cn)])
            pltpu.sync_copy(bd, gd_h.at[pl.ds(base, cn)])
            pltpu.sync_copy(bv, gv_h.at[pl.ds(base, cn)])

        @pl.loop(0, nfull)
        def _(kk):
            chunk(ebase + kk * 128, 128, sidx, didx, bs, bd, bv)

        if tail:
            chunk(ebase + nfull * 128, tail, tsidx, tdidx, tbs, tbd, tbv)

    return k(ns, nv2d, src, dst)


def _scatter_sum(src, arrs, zeros_n, n):
    E = src.shape[0]
    epc = E // _NS
    nfull = epc // 128
    tail = epc - nfull * 128
    rpt = n // _NS
    assert rpt % 8 == 0 and rpt * _NS == n

    mesh = plsc.VectorSubcoreMesh(core_axis_name="c", subcore_axis_name="s")

    @functools.partial(
        pl.kernel,
        out_type=tuple(jax.ShapeDtypeStruct((n, 128), jnp.float32)
                       for _ in range(8)),
        mesh=mesh,
        scratch_types=[
            pltpu.VMEM((128,), jnp.int32),
            pltpu.VMEM((max(tail, 8),), jnp.int32),
            pltpu.VMEM((128, 128), jnp.float32),
            pltpu.VMEM((max(tail, 8), 128), jnp.float32),
            pltpu.VMEM_SHARED((n, 128), jnp.float32),
        ],
    )
    def k(src_h, a0, a1, a2, a3, a4, a5, a6, a7, z_h,
          o0, o1, o2, o3, o4, o5, o6, o7,
          idx_v, idxt_v, rows_v, rowst_v, acc_sh):
        cid = lax.axis_index("c")
        sid = lax.axis_index("s")
        ebase = sid * epc

        def process(a_h, o_h):
            pltpu.sync_copy(z_h.at[pl.ds(sid * rpt, rpt)],
                            acc_sh.at[pl.ds(sid * rpt, rpt)])
            plsc.subcore_barrier()

            @pl.loop(0, nfull)
            def _(kk):
                base = ebase + kk * 128
                pltpu.sync_copy(src_h.at[pl.ds(base, 128)], idx_v)
                pltpu.sync_copy(a_h.at[pl.ds(base, 128)], rows_v)
                pltpu.sync_copy(rows_v, acc_sh.at[idx_v], add=True)

            if tail:
                tbase = ebase + nfull * 128
                pltpu.sync_copy(src_h.at[pl.ds(tbase, tail)],
                                idxt_v.at[pl.ds(0, tail)])
                pltpu.sync_copy(a_h.at[pl.ds(tbase, tail)],
                                rowst_v.at[pl.ds(0, tail)])
                pltpu.sync_copy(rowst_v.at[pl.ds(0, tail)],
                                acc_sh.at[idxt_v.at[pl.ds(0, tail)]], add=True)

            plsc.subcore_barrier()
            pltpu.sync_copy(acc_sh.at[pl.ds(sid * rpt, rpt)],
                            o_h.at[pl.ds(sid * rpt, rpt)])
            plsc.subcore_barrier()

        ins = [a0, a1, a2, a3, a4, a5, a6, a7]
        outs = [o0, o1, o2, o3, o4, o5, o6, o7]
        for g in range(4):
            @pl.when(cid == 0)
            def _():
                process(ins[g], outs[g])

            @pl.when(cid == 1)
            def _():
                process(ins[4 + g], outs[4 + g])

    return k(src, *arrs, zeros_n)


def _scatter_maxmin(src, envc, n):
    E = src.shape[0]
    NW = _NC * _NS
    npt = -(-n // (NW * 8)) * 8
    n_pad = NW * npt
    CHK = 2000
    nchk = E // CHK
    LR = 512
    MB = 16 * LR
    DUMP = npt
    AROWS = npt + 8

    mesh = plsc.VectorSubcoreMesh(core_axis_name="c", subcore_axis_name="s")

    out_types = tuple(jax.ShapeDtypeStruct((n_pad * 16,), jnp.float32)
                      for _ in range(16))
    acc_types = [pltpu.VMEM((AROWS * 16,), jnp.float32) for _ in range(16)]

    @functools.partial(
        pl.kernel,
        out_type=out_types,
        mesh=mesh,
        compiler_params=pltpu.CompilerParams(needs_layout_passes=False),
        scratch_types=acc_types + [
            pltpu.VMEM((CHK,), jnp.int32),
            pltpu.VMEM((MB // 128, 128), jnp.int32),
            pltpu.VMEM((MB,), jnp.int32),
            pltpu.VMEM((128, 128), jnp.float32),
            pltpu.SemaphoreType.DMA,
        ],
    )
    def k(src_h, envc_h, *refs):
        outs = refs[:16]
        accs = refs[16:32]
        srcv, meid, mloc, rows_v, sem = refs[32:]
        amax = accs[:8]
        amin = accs[8:]
        cid = lax.axis_index("c")
        sid = lax.axis_index("s")
        wid = cid * _NS + sid
        nbase = wid * npt

        iota = lax.iota(jnp.int32, 16)
        pinf = jnp.full((16,), jnp.inf, jnp.float32)
        z16 = jnp.zeros((16,), jnp.int32)
        dump16 = jnp.full((16,), DUMP, jnp.int32)

        @pl.loop(0, AROWS)
        def _(i):
            for kk in range(8):
                amax[kk][pl.ds(i * 16, 16)] = -pinf
                amin[kk][pl.ds(i * 16, 16)] = pinf

        @pl.loop(0, MB // 128)
        def _(i):
            for kk in range(8):
                meid[i, pl.ds(kk * 16, 16)] = z16

        @pl.loop(0, MB // 16)
        def _(i):
            mloc[pl.ds(i * 16, 16)] = dump16

        lane_base = iota * LR

        def scan_chunk(c, cnt):
            pltpu.sync_copy(src_h.at[pl.ds(c * CHK, CHK)], srcv)

            def vstep(i, cnt):
                v = srcv[pl.ds(i * 16, 16)]
                m = (v >= nbase) & (v < nbase + npt)
                eid = c * CHK + i * 16 + iota
                idx_dst = lane_base + cnt
                plsc.store_scatter(meid, [idx_dst >> 7, idx_dst & 127], eid,
                                   mask=m)
                plsc.store_scatter(mloc, [idx_dst], v - nbase, mask=m)
                return cnt + jnp.where(m, 1, 0)

            return lax.fori_loop(0, CHK // 16, vstep, cnt)

        cnt = lax.fori_loop(0, nchk, scan_chunk, z16)

        del cnt

        @pl.loop(0, MB // 128)
        def _(b):
            pltpu.async_copy(envc_h.at[meid.at[b]], rows_v, sem).wait()

            @pl.loop(0, 8)
            def _(g):
                gb = g * 16
                nlv = mloc[pl.ds(b * 128 + gb, 16)]
                for t in range(16):
                    ab = nlv[t] * 16
                    for kk in range(8):
                        r = rows_v[gb + t, pl.ds(kk * 16, 16)]
                        amax[kk][pl.ds(ab, 16)] = jnp.maximum(
                            amax[kk][pl.ds(ab, 16)], r)
                        amin[kk][pl.ds(ab, 16)] = jnp.minimum(
                            amin[kk][pl.ds(ab, 16)], r)

        for kk in range(8):
            pltpu.sync_copy(amax[kk].at[pl.ds(0, npt * 16)],
                            outs[kk].at[pl.ds(nbase * 16, npt * 16)])
            pltpu.sync_copy(amin[kk].at[pl.ds(0, npt * 16)],
                            outs[8 + kk].at[pl.ds(nbase * 16, npt * 16)])

    res = k(src, envc)
    omax = jnp.concatenate([r.reshape(n_pad, 16) for r in res[:8]], axis=1)
    omin = jnp.concatenate([r.reshape(n_pad, 16) for r in res[8:]], axis=1)
    return omax, omin


def _node_body(e1, e2, e3, ns, nv,
               eW1a, eW1b, eW1c, eb1, eW2, eb2, uVW, uVb, sW1, sb1, sW2, sb2,
               env, mvs):
    e2f = jnp.where(jnp.isfinite(e2[...]), e2[...], 0.0)
    e3f = jnp.where(jnp.isfinite(e3[...]), e3[...], 0.0)
    h = _silu(jnp.dot(e1[...], eW1a[...], preferred_element_type=jnp.float32)
              + jnp.dot(e2f, eW1b[...], preferred_element_type=jnp.float32)
              + jnp.dot(e3f, eW1c[...], preferred_element_type=jnp.float32)
              + eb1[...])
    env[...] = jnp.dot(h, eW2[...], preferred_element_type=jnp.float32) + eb2[...]
    nvv = nv[...]
    sq = jnp.zeros_like(e2f)
    for c in range(3):
        V = jnp.dot(nvv[:, c * 128:(c + 1) * 128], uVW[...],
                    preferred_element_type=jnp.float32) + uVb[...]
        sq = sq + V * V
    norm = jnp.sqrt(sq)
    gate = _silu(jnp.dot(ns[...], sW1[...], preferred_element_type=jnp.float32)
                 + sb1[...])
    gate = jnp.dot(gate, sW2[...], preferred_element_type=jnp.float32) + sb2[...]
    mvs[...] = gate * norm


def _node_compute(e1, e2, e3, node_scalar, nv2d, params):
    N = node_scalar.shape[0]
    grid = N // NB
    eW1a = params['env_W1'][0:128]
    eW1b = params['env_W1'][128:256]
    eW1c = params['env_W1'][256:384]
    args = (e1, e2, e3, node_scalar, nv2d,
            eW1a, eW1b, eW1c, params['env_b1'][None, :],
            params['env_W2'], params['env_b2'][None, :],
            params['uV_W'], params['uV_b'][None, :],
            params['svg_W1'], params['svg_b1'][None, :],
            params['svg_W2'], params['svg_b2'][None, :])
    nb = lambda d: pl.BlockSpec((NB, d), lambda i: (i, 0))
    wb = lambda a: pl.BlockSpec(a.shape, lambda i: tuple(0 for _ in a.shape))
    out_shapes = (
        jax.ShapeDtypeStruct((N, 128), jnp.float32),
        jax.ShapeDtypeStruct((N, 128), jnp.float32),
    )
    return pl.pallas_call(
        _node_body,
        grid=(grid,),
        in_specs=[nb(128), nb(128), nb(128), nb(128), nb(384)]
                 + [wb(a) for a in args[5:]],
        out_specs=(nb(128), nb(128)),
        out_shape=out_shapes,
    )(*args)


def kernel(node_scalar, node_chiral, node_vector, edge_index, edge_diff,
           edge_dist, triplet_index, pos, params):
    n = node_scalar.shape[0]
    E = edge_index.shape[0]
    src = jnp.asarray(edge_index[:, 0], jnp.int32)
    dst = jnp.asarray(edge_index[:, 1], jnp.int32)
    nv2d = node_vector.reshape(n, 384)

    gsrc, gdst, gvec = _gather(node_scalar, nv2d, src, dst)

    dist2d = edge_dist[:, None]
    ed0 = edge_diff[:, 0:1]
    ed1 = edge_diff[:, 1:2]
    ed2 = edge_diff[:, 2:3]

    mss, envc, vv0, vv1, vv2, ev0, ev1, ev2 = _edge_compute(
        gsrc, gdst, gvec, dist2d, ed0, ed1, ed2, params)

    src_c = src
    n_pad = -(-n // (_NC * _NS * 8)) * (_NC * _NS * 8)
    zeros_n = jnp.zeros((n_pad, 128), jnp.float32)
    sums = _scatter_sum(
        src_c, [mss, envc, vv0, vv1, vv2, ev0, ev1, ev2], zeros_n, n_pad)
    (message_ss, e1, o_vv0, o_vv1, o_vv2,
     o_ev0, o_ev1, o_ev2) = (a[:n] for a in sums)
    e2p, e3p = _scatter_maxmin(src_c, envc, n)
    e2 = e2p[:n]
    e3 = e3p[:n]

    env, mvs = _node_compute(e1, e2, e3, node_scalar, nv2d, params)

    message_vv = jnp.stack([o_vv0, o_vv1, o_vv2], axis=1)
    message_ev = jnp.stack([o_ev0, o_ev1, o_ev2], axis=1)
    return (message_ss, message_vv, message_ev, mvs, env)


# speedup vs baseline: 1.6020x; 1.6020x over previous
import functools
import math

import jax
import jax.numpy as jnp
from jax import lax
from jax.experimental import pallas as pl
from jax.experimental.pallas import tpu as pltpu
from jax.experimental.pallas import tpu_sc as plsc

NODE_SIZE = 128
NUM_RADIAL = 5
CUTOFF = 5.0

_NC = 2
_NS = 16

EB = 1000
NB = 1000


def _silu(x):
    return x * jax.nn.sigmoid(x)


def _softmax(x):
    m = jnp.max(x, axis=1, keepdims=True)
    e = jnp.exp(x - m)
    return e / jnp.sum(e, axis=1, keepdims=True)


def _edge_body(gsrc, gdst, gvec, dist, ed0, ed1, ed2,
               W1s, W1d, b1, W2, b2, sfW, sfb,
               mss, envc, vv0, vv1, vv2, ev0, ev1, ev2):
    h = _silu(jnp.dot(gsrc[...], W1s[...], preferred_element_type=jnp.float32)
              + jnp.dot(gdst[...], W1d[...], preferred_element_type=jnp.float32)
              + b1[...])
    mg = jnp.dot(h, W2[...], preferred_element_type=jnp.float32) + b2[...]
    d = dist[...]
    s = jnp.broadcast_to(sfb[...], mg.shape)
    inv_d = 1.0 / d
    for k in range(NUM_RADIAL):
        fk = (k + 1) * math.pi / CUTOFF
        s = s + (jnp.sin(d * fk) * inv_d) * sfW[k:k + 1, :]
    cut = jnp.where(d < CUTOFF, 0.5 * (jnp.cos(d * (math.pi / CUTOFF)) + 1.0), 0.0)
    mg = mg * (s * cut)
    gate_vv = _softmax(mg[:, 0:128])
    gate_ev = _softmax(mg[:, 128:256])
    mss[...] = mg[:, 256:384]
    envc[...] = mg[:, 384:512]
    gv = gvec[...]
    vv0[...] = gate_vv * gv[:, 0:128]
    vv1[...] = gate_vv * gv[:, 128:256]
    vv2[...] = gate_vv * gv[:, 256:384]
    ev0[...] = gate_ev * ed0[...]
    ev1[...] = gate_ev * ed1[...]
    ev2[...] = gate_ev * ed2[...]


def _edge_compute(gsrc, gdst, gvec, dist2d, ed0, ed1, ed2, params):
    E = gsrc.shape[0]
    grid = E // EB
    W1s = params['sg_W1'][0:128]
    W1d = params['sg_W1'][128:256]
    b1 = params['sg_b1'][None, :]
    b2 = params['sg_b2'][None, :]
    sfb = params['sf_b'][None, :]
    eb = lambda d: pl.BlockSpec((EB, d), lambda i: (i, 0))
    wb = lambda a: pl.BlockSpec(a.shape, lambda i: tuple(0 for _ in a.shape))
    out_shapes = tuple(jax.ShapeDtypeStruct((E, 128), jnp.float32)
                       for _ in range(8))
    return pl.pallas_call(
        _edge_body,
        grid=(grid,),
        in_specs=[eb(128), eb(128), eb(384), eb(1), eb(1), eb(1), eb(1),
                  wb(W1s), wb(W1d), wb(b1), wb(params['sg_W2']), wb(b2),
                  wb(params['sf_W']), wb(sfb)],
        out_specs=tuple(eb(128) for _ in range(8)),
        out_shape=out_shapes,
    )(gsrc, gdst, gvec, dist2d, ed0, ed1, ed2,
      W1s, W1d, b1, params['sg_W2'], b2, params['sf_W'], sfb)


def _gather(ns, nv2d, src, dst):
    N = ns.shape[0]
    E = src.shape[0]
    NW = _NC * _NS
    ept = E // NW
    nfull = ept // 128
    tail = ept - nfull * 128
    assert ept * NW == E and tail % 8 == 0

    mesh = plsc.VectorSubcoreMesh(core_axis_name="c", subcore_axis_name="s")

    @functools.partial(
        pl.kernel,
        out_type=(jax.ShapeDtypeStruct((E, 128), jnp.float32),
                  jax.ShapeDtypeStruct((E, 128), jnp.float32),
                  jax.ShapeDtypeStruct((E, 384), jnp.float32)),
        mesh=mesh,
        scratch_types=[
            pltpu.VMEM((128,), jnp.int32),
            pltpu.VMEM((128,), jnp.int32),
            pltpu.VMEM((128, 128), jnp.float32),
            pltpu.VMEM((128, 128), jnp.float32),
            pltpu.VMEM((128, 384), jnp.float32),
            pltpu.VMEM((max(tail, 8),), jnp.int32),
            pltpu.VMEM((max(tail, 8),), jnp.int32),
            pltpu.VMEM((max(tail, 8), 128), jnp.float32),
            pltpu.VMEM((max(tail, 8), 128), jnp.float32),
            pltpu.VMEM((max(tail, 8), 384), jnp.float32),
            pltpu.SemaphoreType.DMA,
        ],
    )
    def k(ns_h, nv_h, src_h, dst_h, gs_h, gd_h, gv_h,
          sidx, didx, bs, bd, bv, tsidx, tdidx, tbs, tbd, tbv, sem):
        cid = lax.axis_index("c")
        sid = lax.axis_index("s")
        ebase = (cid * _NS + sid) * ept

        def chunk(base, cn, sidx, didx, bs, bd, bv):
            pltpu.sync_copy(src_h.at[pl.ds(base, cn)], sidx)
            pltpu.sync_copy(dst_h.at[pl.ds(base, cn)], didx)
            a = pltpu.async_copy(ns_h.at[sidx], bs, sem)
            b = pltpu.async_copy(ns_h.at[didx], bd, sem)
            c = pltpu.async_copy(nv_h.at[didx], bv, sem)
            a.wait()
            b.wait()
            c.wait()
            pltpu.sync_copy(bs, gs_h.at[pl.ds(base, cn)])
            pltpu.sync_copy(bd, gd_h.at[pl.ds(base, cn)])
            pltpu.sync_copy(bv, gv_h.at[pl.ds(base, cn)])

        @pl.loop(0, nfull)
        def _(kk):
            chunk(ebase + kk * 128, 128, sidx, didx, bs, bd, bv)

        if tail:
            chunk(ebase + nfull * 128, tail, tsidx, tdidx, tbs, tbd, tbv)

    return k(ns, nv2d, src, dst)


def _scatter_sum(src, arrs, zeros_n, n):
    E = src.shape[0]
    epc = E // _NS
    nfull = epc // 128
    tail = epc - nfull * 128
    rpt = n // _NS
    assert rpt % 8 == 0 and rpt * _NS == n

    mesh = plsc.VectorSubcoreMesh(core_axis_name="c", subcore_axis_name="s")

    @functools.partial(
        pl.kernel,
        out_type=tuple(jax.ShapeDtypeStruct((n, 128), jnp.float32)
                       for _ in range(8)),
        mesh=mesh,
        scratch_types=[
            pltpu.VMEM((128,), jnp.int32),
            pltpu.VMEM((max(tail, 8),), jnp.int32),
            pltpu.VMEM((128, 128), jnp.float32),
            pltpu.VMEM((max(tail, 8), 128), jnp.float32),
            pltpu.VMEM_SHARED((n, 128), jnp.float32),
        ],
    )
    def k(src_h, a0, a1, a2, a3, a4, a5, a6, a7, z_h,
          o0, o1, o2, o3, o4, o5, o6, o7,
          idx_v, idxt_v, rows_v, rowst_v, acc_sh):
        cid = lax.axis_index("c")
        sid = lax.axis_index("s")
        ebase = sid * epc

        def process(a_h, o_h):
            pltpu.sync_copy(z_h.at[pl.ds(sid * rpt, rpt)],
                            acc_sh.at[pl.ds(sid * rpt, rpt)])
            plsc.subcore_barrier()

            @pl.loop(0, nfull)
            def _(kk):
                base = ebase + kk * 128
                pltpu.sync_copy(src_h.at[pl.ds(base, 128)], idx_v)
                pltpu.sync_copy(a_h.at[pl.ds(base, 128)], rows_v)
                pltpu.sync_copy(rows_v, acc_sh.at[idx_v], add=True)

            if tail:
                tbase = ebase + nfull * 128
                pltpu.sync_copy(src_h.at[pl.ds(tbase, tail)],
                                idxt_v.at[pl.ds(0, tail)])
                pltpu.sync_copy(a_h.at[pl.ds(tbase, tail)],
                                rowst_v.at[pl.ds(0, tail)])
                pltpu.sync_copy(rowst_v.at[pl.ds(0, tail)],
                                acc_sh.at[idxt_v.at[pl.ds(0, tail)]], add=True)

            plsc.subcore_barrier()
            pltpu.sync_copy(acc_sh.at[pl.ds(sid * rpt, rpt)],
                            o_h.at[pl.ds(sid * rpt, rpt)])
            plsc.subcore_barrier()

        ins = [a0, a1, a2, a3, a4, a5, a6, a7]
        outs = [o0, o1, o2, o3, o4, o5, o6, o7]
        for g in range(4):
            @pl.when(cid == 0)
            def _():
                process(ins[g], outs[g])

            @pl.when(cid == 1)
            def _():
                process(ins[4 + g], outs[4 + g])

    return k(src, *arrs, zeros_n)


def _scatter_maxmin(src, envc, n):
    E = src.shape[0]
    NW = _NC * _NS
    npt = -(-n // (NW * 8)) * 8
    n_pad = NW * npt
    CHK = 2000
    nchk = E // CHK
    LR = 512
    MB = 16 * LR
    DUMP = npt
    AROWS = npt + 8

    mesh = plsc.VectorSubcoreMesh(core_axis_name="c", subcore_axis_name="s")

    out_types = tuple(jax.ShapeDtypeStruct((n_pad * 16,), jnp.float32)
                      for _ in range(16))
    acc_types = [pltpu.VMEM((AROWS * 16,), jnp.float32) for _ in range(16)]

    @functools.partial(
        pl.kernel,
        out_type=out_types,
        mesh=mesh,
        compiler_params=pltpu.CompilerParams(needs_layout_passes=False),
        scratch_types=acc_types + [
            pltpu.VMEM((CHK,), jnp.int32),
            pltpu.VMEM((MB // 128, 128), jnp.int32),
            pltpu.VMEM((MB,), jnp.int32),
            pltpu.VMEM((128, 128), jnp.float32),
            pltpu.SemaphoreType.DMA,
        ],
    )
    def k(src_h, envc_h, *refs):
        outs = refs[:16]
        accs = refs[16:32]
        srcv, meid, mloc, rows_v, sem = refs[32:]
        amax = accs[:8]
        amin = accs[8:]
        cid = lax.axis_index("c")
        sid = lax.axis_index("s")
        wid = cid * _NS + sid
        nbase = wid * npt

        iota = lax.iota(jnp.int32, 16)
        pinf = jnp.full((16,), jnp.inf, jnp.float32)
        z16 = jnp.zeros((16,), jnp.int32)
        dump16 = jnp.full((16,), DUMP, jnp.int32)

        @pl.loop(0, AROWS)
        def _(i):
            for kk in range(8):
                amax[kk][pl.ds(i * 16, 16)] = -pinf
                amin[kk][pl.ds(i * 16, 16)] = pinf

        @pl.loop(0, MB // 128)
        def _(i):
            for kk in range(8):
                meid[i, pl.ds(kk * 16, 16)] = z16

        @pl.loop(0, MB // 16)
        def _(i):
            mloc[pl.ds(i * 16, 16)] = dump16

        lane_base = iota * LR

        def scan_chunk(c, cnt):
            pltpu.sync_copy(src_h.at[pl.ds(c * CHK, CHK)], srcv)

            def vstep(i, cnt):
                v = srcv[pl.ds(i * 16, 16)]
                m = (v >= nbase) & (v < nbase + npt)
                eid = c * CHK + i * 16 + iota
                idx_dst = lane_base + cnt
                plsc.store_scatter(meid, [idx_dst >> 7, idx_dst & 127], eid,
                                   mask=m)
                plsc.store_scatter(mloc, [idx_dst], v - nbase, mask=m)
                return cnt + jnp.where(m, 1, 0)

            return lax.fori_loop(0, CHK // 16, vstep, cnt)

        cnt = lax.fori_loop(0, nchk, scan_chunk, z16)

        for L in range(16):
            cl = cnt[L]

            @pl.loop(0, (cl + 127) // 128)
            def _(s):
                b = L * (LR // 128) + s
                pltpu.async_copy(envc_h.at[meid.at[b]], rows_v, sem).wait()
                nrows = jnp.minimum(jnp.int32(128), cl - s * 128)

                @pl.loop(0, nrows)
                def _(j):
                    jj = b * 128 + j
                    nl = plsc.load_gather(mloc,
                                          [jnp.zeros((16,), jnp.int32) + jj])
                    ci = nl * 16 + iota
                    for kk in range(8):
                        r = rows_v[j, pl.ds(kk * 16, 16)]
                        a = plsc.load_gather(amax[kk], [ci])
                        plsc.store_scatter(amax[kk], [ci], jnp.maximum(a, r))
                        a2 = plsc.load_gather(amin[kk], [ci])
                        plsc.store_scatter(amin[kk], [ci], jnp.minimum(a2, r))

        for kk in range(8):
            pltpu.sync_copy(amax[kk].at[pl.ds(0, npt * 16)],
                            outs[kk].at[pl.ds(nbase * 16, npt * 16)])
            pltpu.sync_copy(amin[kk].at[pl.ds(0, npt * 16)],
                            outs[8 + kk].at[pl.ds(nbase * 16, npt * 16)])

    res = k(src, envc)
    omax = jnp.concatenate([r.reshape(n_pad, 16) for r in res[:8]], axis=1)
    omin = jnp.concatenate([r.reshape(n_pad, 16) for r in res[8:]], axis=1)
    return omax, omin


def _node_body(e1, e2, e3, ns, nv,
               eW1a, eW1b, eW1c, eb1, eW2, eb2, uVW, uVb, sW1, sb1, sW2, sb2,
               env, mvs):
    e2f = jnp.where(jnp.isfinite(e2[...]), e2[...], 0.0)
    e3f = jnp.where(jnp.isfinite(e3[...]), e3[...], 0.0)
    h = _silu(jnp.dot(e1[...], eW1a[...], preferred_element_type=jnp.float32)
              + jnp.dot(e2f, eW1b[...], preferred_element_type=jnp.float32)
              + jnp.dot(e3f, eW1c[...], preferred_element_type=jnp.float32)
              + eb1[...])
    env[...] = jnp.dot(h, eW2[...], preferred_element_type=jnp.float32) + eb2[...]
    nvv = nv[...]
    sq = jnp.zeros_like(e2f)
    for c in range(3):
        V = jnp.dot(nvv[:, c * 128:(c + 1) * 128], uVW[...],
                    preferred_element_type=jnp.float32) + uVb[...]
        sq = sq + V * V
    norm = jnp.sqrt(sq)
    gate = _silu(jnp.dot(ns[...], sW1[...], preferred_element_type=jnp.float32)
                 + sb1[...])
    gate = jnp.dot(gate, sW2[...], preferred_element_type=jnp.float32) + sb2[...]
    mvs[...] = gate * norm


def _node_compute(e1, e2, e3, node_scalar, nv2d, params):
    N = node_scalar.shape[0]
    grid = N // NB
    eW1a = params['env_W1'][0:128]
    eW1b = params['env_W1'][128:256]
    eW1c = params['env_W1'][256:384]
    args = (e1, e2, e3, node_scalar, nv2d,
            eW1a, eW1b, eW1c, params['env_b1'][None, :],
            params['env_W2'], params['env_b2'][None, :],
            params['uV_W'], params['uV_b'][None, :],
            params['svg_W1'], params['svg_b1'][None, :],
            params['svg_W2'], params['svg_b2'][None, :])
    nb = lambda d: pl.BlockSpec((NB, d), lambda i: (i, 0))
    wb = lambda a: pl.BlockSpec(a.shape, lambda i: tuple(0 for _ in a.shape))
    out_shapes = (
        jax.ShapeDtypeStruct((N, 128), jnp.float32),
        jax.ShapeDtypeStruct((N, 128), jnp.float32),
    )
    return pl.pallas_call(
        _node_body,
        grid=(grid,),
        in_specs=[nb(128), nb(128), nb(128), nb(128), nb(384)]
                 + [wb(a) for a in args[5:]],
        out_specs=(nb(128), nb(128)),
        out_shape=out_shapes,
    )(*args)


def kernel(node_scalar, node_chiral, node_vector, edge_index, edge_diff,
           edge_dist, triplet_index, pos, params):
    n = node_scalar.shape[0]
    E = edge_index.shape[0]
    src = jnp.asarray(edge_index[:, 0], jnp.int32)
    dst = jnp.asarray(edge_index[:, 1], jnp.int32)
    nv2d = node_vector.reshape(n, 384)

    gsrc, gdst, gvec = _gather(node_scalar, nv2d, src, dst)

    dist2d = edge_dist[:, None]
    ed0 = edge_diff[:, 0:1]
    ed1 = edge_diff[:, 1:2]
    ed2 = edge_diff[:, 2:3]

    mss, envc, vv0, vv1, vv2, ev0, ev1, ev2 = _edge_compute(
        gsrc, gdst, gvec, dist2d, ed0, ed1, ed2, params)

    src_c = src
    n_pad = -(-n // (_NC * _NS * 8)) * (_NC * _NS * 8)
    zeros_n = jnp.zeros((n_pad, 128), jnp.float32)
    sums = _scatter_sum(
        src_c, [mss, envc, vv0, vv1, vv2, ev0, ev1, ev2], zeros_n, n_pad)
    (message_ss, e1, o_vv0, o_vv1, o_vv2,
     o_ev0, o_ev1, o_ev2) = (a[:n] for a in sums)
    e2p, e3p = _scatter_maxmin(src_c, envc, n)
    e2 = e2p[:n]
    e3 = e3p[:n]

    env, mvs = _node_compute(e1, e2, e3, node_scalar, nv2d, params)

    message_vv = jnp.stack([o_vv0, o_vv1, o_vv2], axis=1)
    message_ev = jnp.stack([o_ev0, o_ev1, o_ev2], axis=1)
    return (message_ss, message_vv, message_ev, mvs, env)


# speedup vs baseline: 2.2088x; 1.3788x over previous
import functools
import math

import jax
import jax.numpy as jnp
from jax import lax
from jax.experimental import pallas as pl
from jax.experimental.pallas import tpu as pltpu
from jax.experimental.pallas import tpu_sc as plsc

NODE_SIZE = 128
NUM_RADIAL = 5
CUTOFF = 5.0

_NC = 2
_NS = 16

EB = 1000
NB = 1000


def _silu(x):
    return x * jax.nn.sigmoid(x)


def _softmax(x):
    m = jnp.max(x, axis=1, keepdims=True)
    e = jnp.exp(x - m)
    return e / jnp.sum(e, axis=1, keepdims=True)


def _radial_body(dist, c0, c1, c2, c3, c4, c5):
    d = dist[...]
    inv_d = 1.0 / d
    cut = jnp.where(d < CUTOFF,
                    0.5 * (jnp.cos(d * (math.pi / CUTOFF)) + 1.0), 0.0)
    cs = (c0, c1, c2, c3, c4)
    for k in range(NUM_RADIAL):
        fk = (k + 1) * math.pi / CUTOFF
        cs[k][...] = jnp.sin(d * fk) * inv_d * cut
    c5[...] = cut


def _radial(edge_dist):
    E = edge_dist.shape[0]
    R = E // 128
    d2 = edge_dist.reshape(R, 128)
    rb = pl.BlockSpec((R, 128), lambda i: (0, 0))
    outs = pl.pallas_call(
        _radial_body,
        grid=(1,),
        in_specs=[rb],
        out_specs=tuple(rb for _ in range(6)),
        out_shape=tuple(jax.ShapeDtypeStruct((R, 128), jnp.float32)
                        for _ in range(6)),
    )(d2)
    return jnp.stack(outs, axis=-1).reshape(E, 6)


def _edge_body(gsrc, gdst, gvec, c6, ed0, ed1, ed2,
               W1s, W1d, b1, W2, b2, W6,
               mss, envc, vv0, vv1, vv2, ev0, ev1, ev2):
    h = _silu(jnp.dot(gsrc[...], W1s[...], preferred_element_type=jnp.float32)
              + jnp.dot(gdst[...], W1d[...], preferred_element_type=jnp.float32)
              + b1[...])
    mg = jnp.dot(h, W2[...], preferred_element_type=jnp.float32) + b2[...]
    sfw = jnp.dot(c6[...], W6[...], preferred_element_type=jnp.float32)
    mg = mg * sfw
    gate_vv = _softmax(mg[:, 0:128])
    gate_ev = _softmax(mg[:, 128:256])
    mss[...] = mg[:, 256:384]
    envc[...] = mg[:, 384:512]
    gv = gvec[...]
    vv0[...] = gate_vv * gv[:, 0:128]
    vv1[...] = gate_vv * gv[:, 128:256]
    vv2[...] = gate_vv * gv[:, 256:384]
    ev0[...] = gate_ev * ed0[...]
    ev1[...] = gate_ev * ed1[...]
    ev2[...] = gate_ev * ed2[...]


def _edge_compute(gsrc, gdst, gvec, c6, ed0, ed1, ed2, params):
    E = gsrc.shape[0]
    grid = E // EB
    W1s = params['sg_W1'][0:128]
    W1d = params['sg_W1'][128:256]
    b1 = params['sg_b1'][None, :]
    b2 = params['sg_b2'][None, :]
    W6 = jnp.concatenate([params['sf_W'], params['sf_b'][None, :]], axis=0)
    eb = lambda d: pl.BlockSpec((EB, d), lambda i: (i, 0))
    wb = lambda a: pl.BlockSpec(a.shape, lambda i: tuple(0 for _ in a.shape))
    out_shapes = tuple(jax.ShapeDtypeStruct((E, 128), jnp.float32)
                       for _ in range(8))
    return pl.pallas_call(
        _edge_body,
        grid=(grid,),
        in_specs=[eb(128), eb(128), eb(384), eb(6), eb(1), eb(1), eb(1),
                  wb(W1s), wb(W1d), wb(b1), wb(params['sg_W2']), wb(b2),
                  wb(W6)],
        out_specs=tuple(eb(128) for _ in range(8)),
        out_shape=out_shapes,
    )(gsrc, gdst, gvec, c6, ed0, ed1, ed2,
      W1s, W1d, b1, params['sg_W2'], b2, W6)


def _gather(ns, nv2d, src, dst):
    N = ns.shape[0]
    E = src.shape[0]
    NW = _NC * _NS
    ept = E // NW
    nfull = ept // 128
    tail = ept - nfull * 128
    assert ept * NW == E and tail % 8 == 0

    mesh = plsc.VectorSubcoreMesh(core_axis_name="c", subcore_axis_name="s")

    @functools.partial(
        pl.kernel,
        out_type=(jax.ShapeDtypeStruct((E, 128), jnp.float32),
                  jax.ShapeDtypeStruct((E, 128), jnp.float32),
                  jax.ShapeDtypeStruct((E, 384), jnp.float32)),
        mesh=mesh,
        scratch_types=[
            pltpu.VMEM((128,), jnp.int32),
            pltpu.VMEM((128,), jnp.int32),
            pltpu.VMEM((128, 128), jnp.float32),
            pltpu.VMEM((128, 128), jnp.float32),
            pltpu.VMEM((128, 384), jnp.float32),
            pltpu.VMEM((max(tail, 8),), jnp.int32),
            pltpu.VMEM((max(tail, 8),), jnp.int32),
            pltpu.VMEM((max(tail, 8), 128), jnp.float32),
            pltpu.VMEM((max(tail, 8), 128), jnp.float32),
            pltpu.VMEM((max(tail, 8), 384), jnp.float32),
            pltpu.SemaphoreType.DMA,
        ],
    )
    def k(ns_h, nv_h, src_h, dst_h, gs_h, gd_h, gv_h,
          sidx, didx, bs, bd, bv, tsidx, tdidx, tbs, tbd, tbv, sem):
        cid = lax.axis_index("c")
        sid = lax.axis_index("s")
        ebase = (cid * _NS + sid) * ept

        def chunk(base, cn, sidx, didx, bs, bd, bv):
            pltpu.sync_copy(src_h.at[pl.ds(base, cn)], sidx)
            pltpu.sync_copy(dst_h.at[pl.ds(base, cn)], didx)
            a = pltpu.async_copy(ns_h.at[sidx], bs, sem)
            b = pltpu.async_copy(ns_h.at[didx], bd, sem)
            c = pltpu.async_copy(nv_h.at[didx], bv, sem)
            a.wait()
            b.wait()
            c.wait()
            pltpu.sync_copy(bs, gs_h.at[pl.ds(base, cn)])
            pltpu.sync_copy(bd, gd_h.at[pl.ds(base, cn)])
            pltpu.sync_copy(bv, gv_h.at[pl.ds(base, cn)])

        @pl.loop(0, nfull)
        def _(kk):
            chunk(ebase + kk * 128, 128, sidx, didx, bs, bd, bv)

        if tail:
            chunk(ebase + nfull * 128, tail, tsidx, tdidx, tbs, tbd, tbv)

    return k(ns, nv2d, src, dst)


def _scatter_sum(src, arrs, zeros_n, n):
    E = src.shape[0]
    epc = E // _NS
    nfull = epc // 128
    tail = epc - nfull * 128
    rpt = n // _NS
    assert rpt % 8 == 0 and rpt * _NS == n

    mesh = plsc.VectorSubcoreMesh(core_axis_name="c", subcore_axis_name="s")

    @functools.partial(
        pl.kernel,
        out_type=tuple(jax.ShapeDtypeStruct((n, 128), jnp.float32)
                       for _ in range(8)),
        mesh=mesh,
        scratch_types=[
            pltpu.VMEM((128,), jnp.int32),
            pltpu.VMEM((max(tail, 8),), jnp.int32),
            pltpu.VMEM((128, 128), jnp.float32),
            pltpu.VMEM((max(tail, 8), 128), jnp.float32),
            pltpu.VMEM_SHARED((n, 128), jnp.float32),
        ],
    )
    def k(src_h, a0, a1, a2, a3, a4, a5, a6, a7, z_h,
          o0, o1, o2, o3, o4, o5, o6, o7,
          idx_v, idxt_v, rows_v, rowst_v, acc_sh):
        cid = lax.axis_index("c")
        sid = lax.axis_index("s")
        ebase = sid * epc

        def process(a_h, o_h):
            pltpu.sync_copy(z_h.at[pl.ds(sid * rpt, rpt)],
                            acc_sh.at[pl.ds(sid * rpt, rpt)])
            plsc.subcore_barrier()

            @pl.loop(0, nfull)
            def _(kk):
                base = ebase + kk * 128
                pltpu.sync_copy(src_h.at[pl.ds(base, 128)], idx_v)
                pltpu.sync_copy(a_h.at[pl.ds(base, 128)], rows_v)
                pltpu.sync_copy(rows_v, acc_sh.at[idx_v], add=True)

            if tail:
                tbase = ebase + nfull * 128
                pltpu.sync_copy(src_h.at[pl.ds(tbase, tail)],
                                idxt_v.at[pl.ds(0, tail)])
                pltpu.sync_copy(a_h.at[pl.ds(tbase, tail)],
                                rowst_v.at[pl.ds(0, tail)])
                pltpu.sync_copy(rowst_v.at[pl.ds(0, tail)],
                                acc_sh.at[idxt_v.at[pl.ds(0, tail)]], add=True)

            plsc.subcore_barrier()
            pltpu.sync_copy(acc_sh.at[pl.ds(sid * rpt, rpt)],
                            o_h.at[pl.ds(sid * rpt, rpt)])
            plsc.subcore_barrier()

        ins = [a0, a1, a2, a3, a4, a5, a6, a7]
        outs = [o0, o1, o2, o3, o4, o5, o6, o7]
        for g in range(4):
            @pl.when(cid == 0)
            def _():
                process(ins[g], outs[g])

            @pl.when(cid == 1)
            def _():
                process(ins[4 + g], outs[4 + g])

    return k(src, *arrs, zeros_n)


def _scatter_maxmin(src, envc, n):
    E = src.shape[0]
    NW = _NC * _NS
    npt = -(-n // (NW * 8)) * 8
    n_pad = NW * npt
    CHK = 2000
    nchk = E // CHK
    LR = 512
    MB = 16 * LR
    DUMP = npt
    AROWS = npt + 8

    mesh = plsc.VectorSubcoreMesh(core_axis_name="c", subcore_axis_name="s")

    out_types = tuple(jax.ShapeDtypeStruct((n_pad * 16,), jnp.float32)
                      for _ in range(16))
    acc_types = [pltpu.VMEM((AROWS * 16,), jnp.float32) for _ in range(16)]

    @functools.partial(
        pl.kernel,
        out_type=out_types,
        mesh=mesh,
        compiler_params=pltpu.CompilerParams(needs_layout_passes=False),
        scratch_types=acc_types + [
            pltpu.VMEM((CHK,), jnp.int32),
            pltpu.VMEM((MB // 128, 128), jnp.int32),
            pltpu.VMEM((MB,), jnp.int32),
            pltpu.VMEM((128, 128), jnp.float32),
            pltpu.SemaphoreType.DMA,
        ],
    )
    def k(src_h, envc_h, *refs):
        outs = refs[:16]
        accs = refs[16:32]
        srcv, meid, mloc, rows_v, sem = refs[32:]
        amax = accs[:8]
        amin = accs[8:]
        cid = lax.axis_index("c")
        sid = lax.axis_index("s")
        wid = cid * _NS + sid
        nbase = wid * npt

        iota = lax.iota(jnp.int32, 16)
        pinf = jnp.full((16,), jnp.inf, jnp.float32)
        z16 = jnp.zeros((16,), jnp.int32)
        dump16 = jnp.full((16,), DUMP, jnp.int32)

        @pl.loop(0, AROWS)
        def _(i):
            for kk in range(8):
                amax[kk][pl.ds(i * 16, 16)] = -pinf
                amin[kk][pl.ds(i * 16, 16)] = pinf

        @pl.loop(0, MB // 128)
        def _(i):
            for kk in range(8):
                meid[i, pl.ds(kk * 16, 16)] = z16

        @pl.loop(0, MB // 16)
        def _(i):
            mloc[pl.ds(i * 16, 16)] = dump16

        lane_base = iota * LR

        def scan_chunk(c, cnt):
            pltpu.sync_copy(src_h.at[pl.ds(c * CHK, CHK)], srcv)

            def vstep(i, cnt):
                v = srcv[pl.ds(i * 16, 16)]
                m = (v >= nbase) & (v < nbase + npt)
                eid = c * CHK + i * 16 + iota
                idx_dst = lane_base + cnt
                plsc.store_scatter(meid, [idx_dst >> 7, idx_dst & 127], eid,
                                   mask=m)
                plsc.store_scatter(mloc, [idx_dst], v - nbase, mask=m)
                return cnt + jnp.where(m, 1, 0)

            return lax.fori_loop(0, CHK // 16, vstep, cnt)

        cnt = lax.fori_loop(0, nchk, scan_chunk, z16)

        for L in range(16):
            cl = cnt[L]

            @pl.loop(0, (cl + 127) // 128)
            def _(s):
                b = L * (LR // 128) + s
                pltpu.async_copy(envc_h.at[meid.at[b]], rows_v, sem).wait()
                nrows = jnp.minimum(jnp.int32(128), cl - s * 128)

                @pl.loop(0, nrows)
                def _(j):
                    jj = b * 128 + j
                    nl = plsc.load_gather(mloc,
                                          [jnp.zeros((16,), jnp.int32) + jj])
                    ci = nl * 16 + iota
                    for kk in range(8):
                        r = rows_v[j, pl.ds(kk * 16, 16)]
                        a = plsc.load_gather(amax[kk], [ci])
                        plsc.store_scatter(amax[kk], [ci], jnp.maximum(a, r))
                        a2 = plsc.load_gather(amin[kk], [ci])
                        plsc.store_scatter(amin[kk], [ci], jnp.minimum(a2, r))

        for kk in range(8):
            pltpu.sync_copy(amax[kk].at[pl.ds(0, npt * 16)],
                            outs[kk].at[pl.ds(nbase * 16, npt * 16)])
            pltpu.sync_copy(amin[kk].at[pl.ds(0, npt * 16)],
                            outs[8 + kk].at[pl.ds(nbase * 16, npt * 16)])

    res = k(src, envc)
    omax = jnp.concatenate([r.reshape(n_pad, 16) for r in res[:8]], axis=1)
    omin = jnp.concatenate([r.reshape(n_pad, 16) for r in res[8:]], axis=1)
    return omax, omin


def _node_body(e1, e2, e3, ns, nv,
               eW1a, eW1b, eW1c, eb1, eW2, eb2, uVW, uVb, sW1, sb1, sW2, sb2,
               env, mvs):
    e2f = jnp.where(jnp.isfinite(e2[...]), e2[...], 0.0)
    e3f = jnp.where(jnp.isfinite(e3[...]), e3[...], 0.0)
    h = _silu(jnp.dot(e1[...], eW1a[...], preferred_element_type=jnp.float32)
              + jnp.dot(e2f, eW1b[...], preferred_element_type=jnp.float32)
              + jnp.dot(e3f, eW1c[...], preferred_element_type=jnp.float32)
              + eb1[...])
    env[...] = jnp.dot(h, eW2[...], preferred_element_type=jnp.float32) + eb2[...]
    nvv = nv[...]
    sq = jnp.zeros_like(e2f)
    for c in range(3):
        V = jnp.dot(nvv[:, c * 128:(c + 1) * 128], uVW[...],
                    preferred_element_type=jnp.float32) + uVb[...]
        sq = sq + V * V
    norm = jnp.sqrt(sq)
    gate = _silu(jnp.dot(ns[...], sW1[...], preferred_element_type=jnp.float32)
                 + sb1[...])
    gate = jnp.dot(gate, sW2[...], preferred_element_type=jnp.float32) + sb2[...]
    mvs[...] = gate * norm


def _node_compute(e1, e2, e3, node_scalar, nv2d, params):
    N = node_scalar.shape[0]
    grid = N // NB
    eW1a = params['env_W1'][0:128]
    eW1b = params['env_W1'][128:256]
    eW1c = params['env_W1'][256:384]
    args = (e1, e2, e3, node_scalar, nv2d,
            eW1a, eW1b, eW1c, params['env_b1'][None, :],
            params['env_W2'], params['env_b2'][None, :],
            params['uV_W'], params['uV_b'][None, :],
            params['svg_W1'], params['svg_b1'][None, :],
            params['svg_W2'], params['svg_b2'][None, :])
    nb = lambda d: pl.BlockSpec((NB, d), lambda i: (i, 0))
    wb = lambda a: pl.BlockSpec(a.shape, lambda i: tuple(0 for _ in a.shape))
    out_shapes = (
        jax.ShapeDtypeStruct((N, 128), jnp.float32),
        jax.ShapeDtypeStruct((N, 128), jnp.float32),
    )
    return pl.pallas_call(
        _node_body,
        grid=(grid,),
        in_specs=[nb(128), nb(128), nb(128), nb(128), nb(384)]
                 + [wb(a) for a in args[5:]],
        out_specs=(nb(128), nb(128)),
        out_shape=out_shapes,
    )(*args)


def kernel(node_scalar, node_chiral, node_vector, edge_index, edge_diff,
           edge_dist, triplet_index, pos, params):
    n = node_scalar.shape[0]
    E = edge_index.shape[0]
    src = jnp.asarray(edge_index[:, 0], jnp.int32)
    dst = jnp.asarray(edge_index[:, 1], jnp.int32)
    nv2d = node_vector.reshape(n, 384)

    gsrc, gdst, gvec = _gather(node_scalar, nv2d, src, dst)

    c6 = _radial(edge_dist)
    ed0 = edge_diff[:, 0:1]
    ed1 = edge_diff[:, 1:2]
    ed2 = edge_diff[:, 2:3]

    mss, envc, vv0, vv1, vv2, ev0, ev1, ev2 = _edge_compute(
        gsrc, gdst, gvec, c6, ed0, ed1, ed2, params)

    src_c = src
    n_pad = -(-n // (_NC * _NS * 8)) * (_NC * _NS * 8)
    zeros_n = jnp.zeros((n_pad, 128), jnp.float32)
    sums = _scatter_sum(
        src_c, [mss, envc, vv0, vv1, vv2, ev0, ev1, ev2], zeros_n, n_pad)
    (message_ss, e1, o_vv0, o_vv1, o_vv2,
     o_ev0, o_ev1, o_ev2) = (a[:n] for a in sums)
    e2p, e3p = _scatter_maxmin(src_c, envc, n)
    e2 = e2p[:n]
    e3 = e3p[:n]

    env, mvs = _node_compute(e1, e2, e3, node_scalar, nv2d, params)

    message_vv = jnp.stack([o_vv0, o_vv1, o_vv2], axis=1)
    message_ev = jnp.stack([o_ev0, o_ev1, o_ev2], axis=1)
    return (message_ss, message_vv, message_ev, mvs, env)


# speedup vs baseline: 2.4441x; 1.1065x over previous
import functools
import math

import jax
import jax.numpy as jnp
from jax import lax
from jax.experimental import pallas as pl
from jax.experimental.pallas import tpu as pltpu
from jax.experimental.pallas import tpu_sc as plsc

NODE_SIZE = 128
NUM_RADIAL = 5
CUTOFF = 5.0

_NC = 2
_NS = 16

EB = 1000
NB = 1000


def _silu(x):
    return x * jax.nn.sigmoid(x)


def _softmax(x):
    m = jnp.max(x, axis=1, keepdims=True)
    e = jnp.exp(x - m)
    return e / jnp.sum(e, axis=1, keepdims=True)


def _radial_body(dist, c0, c1, c2, c3, c4, c5):
    d = dist[...]
    inv_d = 1.0 / d
    cut = jnp.where(d < CUTOFF,
                    0.5 * (jnp.cos(d * (math.pi / CUTOFF)) + 1.0), 0.0)
    cs = (c0, c1, c2, c3, c4)
    for k in range(NUM_RADIAL):
        fk = (k + 1) * math.pi / CUTOFF
        cs[k][...] = jnp.sin(d * fk) * inv_d * cut
    c5[...] = cut


def _radial(edge_dist):
    E = edge_dist.shape[0]
    R = E // 128
    d2 = edge_dist.reshape(R, 128)
    rb = pl.BlockSpec((R, 128), lambda i: (0, 0))
    outs = pl.pallas_call(
        _radial_body,
        grid=(1,),
        in_specs=[rb],
        out_specs=tuple(rb for _ in range(6)),
        out_shape=tuple(jax.ShapeDtypeStruct((R, 128), jnp.float32)
                        for _ in range(6)),
    )(d2)
    return jnp.stack(outs, axis=-1).reshape(E, 6)


def _edge_body(gsrc, gdst, gvec, c6, ed0, ed1, ed2,
               W1s, W1d, b1, W2, b2, W6,
               mss, envc, vv0, vv1, vv2, ev0, ev1, ev2):
    h = _silu(jnp.dot(gsrc[...], W1s[...], preferred_element_type=jnp.float32)
              + jnp.dot(gdst[...], W1d[...], preferred_element_type=jnp.float32)
              + b1[...])
    mg = jnp.dot(h, W2[...], preferred_element_type=jnp.float32) + b2[...]
    sfw = jnp.dot(c6[...], W6[...], preferred_element_type=jnp.float32)
    mg = mg * sfw
    gate_vv = _softmax(mg[:, 0:128])
    gate_ev = _softmax(mg[:, 128:256])
    mss[...] = mg[:, 256:384]
    envc[...] = mg[:, 384:512]
    gv = gvec[...]
    vv0[...] = gate_vv * gv[:, 0:128]
    vv1[...] = gate_vv * gv[:, 128:256]
    vv2[...] = gate_vv * gv[:, 256:384]
    ev0[...] = gate_ev * ed0[...]
    ev1[...] = gate_ev * ed1[...]
    ev2[...] = gate_ev * ed2[...]


def _edge_compute(gsrc, gdst, gvec, c6, ed0, ed1, ed2, params):
    E = gsrc.shape[0]
    grid = E // EB
    W1s = params['sg_W1'][0:128]
    W1d = params['sg_W1'][128:256]
    b1 = params['sg_b1'][None, :]
    b2 = params['sg_b2'][None, :]
    W6 = jnp.concatenate([params['sf_W'], params['sf_b'][None, :]], axis=0)
    eb = lambda d: pl.BlockSpec((EB, d), lambda i: (i, 0))
    wb = lambda a: pl.BlockSpec(a.shape, lambda i: tuple(0 for _ in a.shape))
    out_shapes = tuple(jax.ShapeDtypeStruct((E, 128), jnp.float32)
                       for _ in range(8))
    return pl.pallas_call(
        _edge_body,
        grid=(grid,),
        in_specs=[eb(128), eb(128), eb(384), eb(6), eb(1), eb(1), eb(1),
                  wb(W1s), wb(W1d), wb(b1), wb(params['sg_W2']), wb(b2),
                  wb(W6)],
        out_specs=tuple(eb(128) for _ in range(8)),
        out_shape=out_shapes,
    )(gsrc, gdst, gvec, c6, ed0, ed1, ed2,
      W1s, W1d, b1, params['sg_W2'], b2, W6)


def _gather(ns, nv2d, src, dst):
    N = ns.shape[0]
    E = src.shape[0]
    NW = _NC * _NS
    ept = E // NW
    nfull = ept // 128
    tail = ept - nfull * 128
    assert ept * NW == E and tail % 8 == 0

    mesh = plsc.VectorSubcoreMesh(core_axis_name="c", subcore_axis_name="s")

    @functools.partial(
        pl.kernel,
        out_type=(jax.ShapeDtypeStruct((E, 128), jnp.float32),
                  jax.ShapeDtypeStruct((E, 128), jnp.float32),
                  jax.ShapeDtypeStruct((E, 384), jnp.float32)),
        mesh=mesh,
        scratch_types=[
            pltpu.VMEM((128,), jnp.int32),
            pltpu.VMEM((128,), jnp.int32),
            pltpu.VMEM((128, 128), jnp.float32),
            pltpu.VMEM((128, 128), jnp.float32),
            pltpu.VMEM((128, 384), jnp.float32),
            pltpu.VMEM((max(tail, 8),), jnp.int32),
            pltpu.VMEM((max(tail, 8),), jnp.int32),
            pltpu.VMEM((max(tail, 8), 128), jnp.float32),
            pltpu.VMEM((max(tail, 8), 128), jnp.float32),
            pltpu.VMEM((max(tail, 8), 384), jnp.float32),
            pltpu.SemaphoreType.DMA,
        ],
    )
    def k(ns_h, nv_h, src_h, dst_h, gs_h, gd_h, gv_h,
          sidx, didx, bs, bd, bv, tsidx, tdidx, tbs, tbd, tbv, sem):
        cid = lax.axis_index("c")
        sid = lax.axis_index("s")
        ebase = (cid * _NS + sid) * ept

        def chunk(base, cn, sidx, didx, bs, bd, bv):
            pltpu.sync_copy(src_h.at[pl.ds(base, cn)], sidx)
            pltpu.sync_copy(dst_h.at[pl.ds(base, cn)], didx)
            a = pltpu.async_copy(ns_h.at[sidx], bs, sem)
            b = pltpu.async_copy(ns_h.at[didx], bd, sem)
            c = pltpu.async_copy(nv_h.at[didx], bv, sem)
            a.wait()
            b.wait()
            c.wait()
            pltpu.sync_copy(bs, gs_h.at[pl.ds(base, cn)])
            pltpu.sync_copy(bd, gd_h.at[pl.ds(base, cn)])
            pltpu.sync_copy(bv, gv_h.at[pl.ds(base, cn)])

        @pl.loop(0, nfull)
        def _(kk):
            chunk(ebase + kk * 128, 128, sidx, didx, bs, bd, bv)

        if tail:
            chunk(ebase + nfull * 128, tail, tsidx, tdidx, tbs, tbd, tbv)

    return k(ns, nv2d, src, dst)


def _scatter_sum(src, arrs, zeros_n, n):
    E = src.shape[0]
    epc = E // _NS
    nfull = epc // 128
    tail = epc - nfull * 128
    rpt = n // _NS
    assert rpt % 8 == 0 and rpt * _NS == n

    mesh = plsc.VectorSubcoreMesh(core_axis_name="c", subcore_axis_name="s")

    @functools.partial(
        pl.kernel,
        out_type=tuple(jax.ShapeDtypeStruct((n, 128), jnp.float32)
                       for _ in range(8)),
        mesh=mesh,
        scratch_types=[
            pltpu.VMEM((128,), jnp.int32),
            pltpu.VMEM((128,), jnp.int32),
            pltpu.VMEM((max(tail, 8),), jnp.int32),
            pltpu.VMEM((128, 128), jnp.float32),
            pltpu.VMEM((128, 128), jnp.float32),
            pltpu.VMEM((max(tail, 8), 128), jnp.float32),
            pltpu.VMEM_SHARED((n, 128), jnp.float32),
            pltpu.SemaphoreType.DMA,
            pltpu.SemaphoreType.DMA,
            pltpu.SemaphoreType.DMA,
            pltpu.SemaphoreType.DMA,
        ],
    )
    def k(src_h, a0, a1, a2, a3, a4, a5, a6, a7, z_h,
          o0, o1, o2, o3, o4, o5, o6, o7,
          idx0, idx1, idxt_v, rows0, rows1, rowst_v, acc_sh,
          semg0, semg1, sems0, sems1):
        cid = lax.axis_index("c")
        sid = lax.axis_index("s")
        ebase = sid * epc
        npairs = nfull // 2
        assert nfull == 2 * npairs

        def process(a_h, o_h):
            pltpu.sync_copy(z_h.at[pl.ds(sid * rpt, rpt)],
                            acc_sh.at[pl.ds(sid * rpt, rpt)])
            plsc.subcore_barrier()

            def issue_gather(c, idxb, rowsb, sem):
                base = ebase + c * 128
                pltpu.async_copy(src_h.at[pl.ds(base, 128)], idxb, sem)
                pltpu.async_copy(a_h.at[pl.ds(base, 128)], rowsb, sem)

            def wait_gather(idxb, rowsb, sem):
                pltpu.make_async_copy(src_h.at[pl.ds(ebase, 128)],
                                      idxb, sem).wait()
                pltpu.make_async_copy(a_h.at[pl.ds(ebase, 128)],
                                      rowsb, sem).wait()

            def issue_scatter(idxb, rowsb, sem):
                pltpu.async_copy(rowsb, acc_sh.at[idxb], sem, add=True)

            def wait_scatter(rowsb, sem):
                pltpu.make_async_copy(a_h.at[pl.ds(ebase, 128)],
                                      rowsb, sem).wait()

            issue_gather(0, idx0, rows0, semg0)

            @pl.loop(0, npairs)
            def _(q):
                c0 = q * 2
                wait_gather(idx0, rows0, semg0)

                @pl.when(q > 0)
                def _():
                    wait_scatter(rows1, sems1)

                issue_gather(c0 + 1, idx1, rows1, semg1)
                issue_scatter(idx0, rows0, sems0)
                wait_gather(idx1, rows1, semg1)
                wait_scatter(rows0, sems0)

                @pl.when(q < npairs - 1)
                def _():
                    issue_gather(c0 + 2, idx0, rows0, semg0)

                issue_scatter(idx1, rows1, sems1)

            wait_scatter(rows1, sems1)

            if tail:
                tbase = ebase + nfull * 128
                pltpu.sync_copy(src_h.at[pl.ds(tbase, tail)],
                                idxt_v.at[pl.ds(0, tail)])
                pltpu.sync_copy(a_h.at[pl.ds(tbase, tail)],
                                rowst_v.at[pl.ds(0, tail)])
                pltpu.sync_copy(rowst_v.at[pl.ds(0, tail)],
                                acc_sh.at[idxt_v.at[pl.ds(0, tail)]], add=True)

            plsc.subcore_barrier()
            pltpu.sync_copy(acc_sh.at[pl.ds(sid * rpt, rpt)],
                            o_h.at[pl.ds(sid * rpt, rpt)])
            plsc.subcore_barrier()

        ins = [a0, a1, a2, a3, a4, a5, a6, a7]
        outs = [o0, o1, o2, o3, o4, o5, o6, o7]
        for g in range(4):
            @pl.when(cid == 0)
            def _():
                process(ins[g], outs[g])

            @pl.when(cid == 1)
            def _():
                process(ins[4 + g], outs[4 + g])

    return k(src, *arrs, zeros_n)


def _scatter_maxmin(src, envc, n):
    E = src.shape[0]
    NW = _NC * _NS
    npt = -(-n // (NW * 8)) * 8
    n_pad = NW * npt
    CHK = 2000
    nchk = E // CHK
    LR = 512
    MB = 16 * LR
    DUMP = npt
    AROWS = npt + 8

    mesh = plsc.VectorSubcoreMesh(core_axis_name="c", subcore_axis_name="s")

    out_types = tuple(jax.ShapeDtypeStruct((n_pad * 16,), jnp.float32)
                      for _ in range(16))
    acc_types = [pltpu.VMEM((AROWS * 16,), jnp.float32) for _ in range(16)]

    @functools.partial(
        pl.kernel,
        out_type=out_types,
        mesh=mesh,
        compiler_params=pltpu.CompilerParams(needs_layout_passes=False),
        scratch_types=acc_types + [
            pltpu.VMEM((CHK,), jnp.int32),
            pltpu.VMEM((MB // 128, 128), jnp.int32),
            pltpu.VMEM((MB,), jnp.int32),
            pltpu.VMEM((128, 128), jnp.float32),
            pltpu.SemaphoreType.DMA,
        ],
    )
    def k(src_h, envc_h, *refs):
        outs = refs[:16]
        accs = refs[16:32]
        srcv, meid, mloc, rows_v, sem = refs[32:]
        amax = accs[:8]
        amin = accs[8:]
        cid = lax.axis_index("c")
        sid = lax.axis_index("s")
        wid = cid * _NS + sid
        nbase = wid * npt

        iota = lax.iota(jnp.int32, 16)
        pinf = jnp.full((16,), jnp.inf, jnp.float32)
        z16 = jnp.zeros((16,), jnp.int32)
        dump16 = jnp.full((16,), DUMP, jnp.int32)

        @pl.loop(0, AROWS)
        def _(i):
            for kk in range(8):
                amax[kk][pl.ds(i * 16, 16)] = -pinf
                amin[kk][pl.ds(i * 16, 16)] = pinf

        @pl.loop(0, MB // 128)
        def _(i):
            for kk in range(8):
                meid[i, pl.ds(kk * 16, 16)] = z16

        @pl.loop(0, MB // 16)
        def _(i):
            mloc[pl.ds(i * 16, 16)] = dump16

        lane_base = iota * LR

        def scan_chunk(c, cnt):
            pltpu.sync_copy(src_h.at[pl.ds(c * CHK, CHK)], srcv)

            def vstep(i, cnt):
                v = srcv[pl.ds(i * 16, 16)]
                m = (v >= nbase) & (v < nbase + npt)
                eid = c * CHK + i * 16 + iota
                idx_dst = lane_base + cnt
                plsc.store_scatter(meid, [idx_dst >> 7, idx_dst & 127], eid,
                                   mask=m)
                plsc.store_scatter(mloc, [idx_dst], v - nbase, mask=m)
                return cnt + jnp.where(m, 1, 0)

            return lax.fori_loop(0, CHK // 16, vstep, cnt)

        cnt = lax.fori_loop(0, nchk, scan_chunk, z16)

        for L in range(16):
            cl = cnt[L]

            @pl.loop(0, (cl + 127) // 128)
            def _(s):
                b = L * (LR // 128) + s
                pltpu.async_copy(envc_h.at[meid.at[b]], rows_v, sem).wait()
                nrows = jnp.minimum(jnp.int32(128), cl - s * 128)

                @pl.loop(0, nrows)
                def _(j):
                    jj = b * 128 + j
                    nl = plsc.load_gather(mloc,
                                          [jnp.zeros((16,), jnp.int32) + jj])
                    ci = nl * 16 + iota
                    for kk in range(8):
                        r = rows_v[j, pl.ds(kk * 16, 16)]
                        a = plsc.load_gather(amax[kk], [ci])
                        plsc.store_scatter(amax[kk], [ci], jnp.maximum(a, r))
                        a2 = plsc.load_gather(amin[kk], [ci])
                        plsc.store_scatter(amin[kk], [ci], jnp.minimum(a2, r))

        for kk in range(8):
            pltpu.sync_copy(amax[kk].at[pl.ds(0, npt * 16)],
                            outs[kk].at[pl.ds(nbase * 16, npt * 16)])
            pltpu.sync_copy(amin[kk].at[pl.ds(0, npt * 16)],
                            outs[8 + kk].at[pl.ds(nbase * 16, npt * 16)])

    res = k(src, envc)
    omax = jnp.concatenate([r.reshape(n_pad, 16) for r in res[:8]], axis=1)
    omin = jnp.concatenate([r.reshape(n_pad, 16) for r in res[8:]], axis=1)
    return omax, omin


def _node_body(e1, e2, e3, ns, nv,
               eW1a, eW1b, eW1c, eb1, eW2, eb2, uVW, uVb, sW1, sb1, sW2, sb2,
               env, mvs):
    e2f = jnp.where(jnp.isfinite(e2[...]), e2[...], 0.0)
    e3f = jnp.where(jnp.isfinite(e3[...]), e3[...], 0.0)
    h = _silu(jnp.dot(e1[...], eW1a[...], preferred_element_type=jnp.float32)
              + jnp.dot(e2f, eW1b[...], preferred_element_type=jnp.float32)
              + jnp.dot(e3f, eW1c[...], preferred_element_type=jnp.float32)
              + eb1[...])
    env[...] = jnp.dot(h, eW2[...], preferred_element_type=jnp.float32) + eb2[...]
    nvv = nv[...]
    sq = jnp.zeros_like(e2f)
    for c in range(3):
        V = jnp.dot(nvv[:, c * 128:(c + 1) * 128], uVW[...],
                    preferred_element_type=jnp.float32) + uVb[...]
        sq = sq + V * V
    norm = jnp.sqrt(sq)
    gate = _silu(jnp.dot(ns[...], sW1[...], preferred_element_type=jnp.float32)
                 + sb1[...])
    gate = jnp.dot(gate, sW2[...], preferred_element_type=jnp.float32) + sb2[...]
    mvs[...] = gate * norm


def _node_compute(e1, e2, e3, node_scalar, nv2d, params):
    N = node_scalar.shape[0]
    grid = N // NB
    eW1a = params['env_W1'][0:128]
    eW1b = params['env_W1'][128:256]
    eW1c = params['env_W1'][256:384]
    args = (e1, e2, e3, node_scalar, nv2d,
            eW1a, eW1b, eW1c, params['env_b1'][None, :],
            params['env_W2'], params['env_b2'][None, :],
            params['uV_W'], params['uV_b'][None, :],
            params['svg_W1'], params['svg_b1'][None, :],
            params['svg_W2'], params['svg_b2'][None, :])
    nb = lambda d: pl.BlockSpec((NB, d), lambda i: (i, 0))
    wb = lambda a: pl.BlockSpec(a.shape, lambda i: tuple(0 for _ in a.shape))
    out_shapes = (
        jax.ShapeDtypeStruct((N, 128), jnp.float32),
        jax.ShapeDtypeStruct((N, 128), jnp.float32),
    )
    return pl.pallas_call(
        _node_body,
        grid=(grid,),
        in_specs=[nb(128), nb(128), nb(128), nb(128), nb(384)]
                 + [wb(a) for a in args[5:]],
        out_specs=(nb(128), nb(128)),
        out_shape=out_shapes,
    )(*args)


def kernel(node_scalar, node_chiral, node_vector, edge_index, edge_diff,
           edge_dist, triplet_index, pos, params):
    n = node_scalar.shape[0]
    E = edge_index.shape[0]
    src = jnp.asarray(edge_index[:, 0], jnp.int32)
    dst = jnp.asarray(edge_index[:, 1], jnp.int32)
    nv2d = node_vector.reshape(n, 384)

    gsrc, gdst, gvec = _gather(node_scalar, nv2d, src, dst)

    c6 = _radial(edge_dist)
    ed0 = edge_diff[:, 0:1]
    ed1 = edge_diff[:, 1:2]
    ed2 = edge_diff[:, 2:3]

    mss, envc, vv0, vv1, vv2, ev0, ev1, ev2 = _edge_compute(
        gsrc, gdst, gvec, c6, ed0, ed1, ed2, params)

    src_c = src
    n_pad = -(-n // (_NC * _NS * 8)) * (_NC * _NS * 8)
    zeros_n = jnp.zeros((n_pad, 128), jnp.float32)
    sums = _scatter_sum(
        src_c, [mss, envc, vv0, vv1, vv2, ev0, ev1, ev2], zeros_n, n_pad)
    (message_ss, e1, o_vv0, o_vv1, o_vv2,
     o_ev0, o_ev1, o_ev2) = (a[:n] for a in sums)
    e2p, e3p = _scatter_maxmin(src_c, envc, n)
    e2 = e2p[:n]
    e3 = e3p[:n]

    env, mvs = _node_compute(e1, e2, e3, node_scalar, nv2d, params)

    message_vv = jnp.stack([o_vv0, o_vv1, o_vv2], axis=1)
    message_ev = jnp.stack([o_ev0, o_ev1, o_ev2], axis=1)
    return (message_ss, message_vv, message_ev, mvs, env)
